# Initial kernel scaffold; baseline (speedup 1.0000x reference)
import jax
import jax.numpy as jnp
from jax.experimental import pallas as pl


def kernel(z, log_tau):
    raise NotImplementedError("write your pallas kernel here")


# speedup vs baseline: 11.2642x; 11.2642x over previous
import numpy as np
import jax
import jax.numpy as jnp
from jax.experimental import pallas as pl
from jax.experimental.pallas import tpu as pltpu

_B = 128
_U = 3
_DP = 32
_K = 32


def _candidate_rank_tables():
    tris = [(a, b, c)
            for a in range(_DP) for b in range(_DP) for c in range(_DP)
            if (a + 1) * (b + 1) * (c + 1) <= _K]
    tris = np.array(tris, dtype=np.int32)
    c = tris.shape[0]
    cpad = ((c + 127) // 128) * 128
    pad = np.full((cpad - c, 3), _DP - 1, np.int32)
    tris = np.concatenate([tris, pad], axis=0)
    return c, cpad, tris


_C, _CPAD, _TRIS = _candidate_rank_tables()


def _body(z_ref, lt_ref, r0_ref, r1_ref, r2_ref, idx_ref, w_ref):
    tau = jnp.exp(lt_ref[0])
    z = z_ref[:, :]
    lane32 = jax.lax.broadcasted_iota(jnp.int32, (_B, _DP), 1)

    svals = []
    sidxs = []
    for u in range(_U):
        x = z[:, u * _DP:(u + 1) * _DP] / tau
        m = jnp.max(x, axis=1, keepdims=True)
        e = jnp.exp(x - m)
        s = jnp.sum(e, axis=1, keepdims=True)
        p = e / s
        su = jnp.zeros((_B, _DP), jnp.float32)
        au = jnp.zeros((_B, _DP), jnp.int32)
        work = p
        for r in range(_DP):
            mv = jnp.max(work, axis=1, keepdims=True)
            mi = jnp.min(jnp.where(work == mv, lane32, _DP * 2),
                         axis=1, keepdims=True)
            su = jnp.where(lane32 == r, mv, su)
            au = jnp.where(lane32 == r, mi, au)
            work = jnp.where(lane32 == mi, -1.0, work)
        svals.append(su)
        sidxs.append(au)

    vs = []
    cs = []
    for u, r_ref in enumerate((r0_ref, r1_ref, r2_ref)):
        ranks = r_ref[:, :]
        vu = jnp.zeros((_B, _CPAD), jnp.float32)
        iu = jnp.zeros((_B, _CPAD), jnp.int32)
        for i in range(_DP):
            msk = ranks == i
            vu = jnp.where(msk, svals[u][:, i:i + 1], vu)
            iu = jnp.where(msk, sidxs[u][:, i:i + 1], iu)
        vs.append(vu)
        cs.append(iu)

    cand_v = (vs[0] * vs[1]) * vs[2]
    comb = cs[0] * (_DP * _DP) + cs[1] * _DP + cs[2]
    clane = jax.lax.broadcasted_iota(jnp.int32, (_B, _CPAD), 1)
    padm = clane >= _C
    cand_v = jnp.where(padm, -1.0, cand_v)
    comb = jnp.where(padm, 1 << 20, comb)

    w_out = jnp.zeros((_B, _K), jnp.float32)
    i_out = jnp.zeros((_B, _K), jnp.int32)
    for t in range(_K):
        mv = jnp.max(cand_v, axis=1, keepdims=True)
        bi = jnp.min(jnp.where(cand_v == mv, comb, 1 << 20),
                     axis=1, keepdims=True)
        w_out = jnp.where(lane32 == t, mv, w_out)
        i_out = jnp.where(lane32 == t, bi, i_out)
        cand_v = jnp.where((cand_v == mv) & (comb == bi), -1.0, cand_v)

    idx_ref[:, :] = i_out
    w_ref[:, :] = w_out


def kernel(z, log_tau):
    r0 = jnp.asarray(_TRIS[:, 0].reshape(1, _CPAD))
    r1 = jnp.asarray(_TRIS[:, 1].reshape(1, _CPAD))
    r2 = jnp.asarray(_TRIS[:, 2].reshape(1, _CPAD))
    indices, weights = pl.pallas_call(
        _body,
        out_shape=[
            jax.ShapeDtypeStruct((_B, _K), jnp.int32),
            jax.ShapeDtypeStruct((_B, _K), jnp.float32),
        ],
        in_specs=[
            pl.BlockSpec(memory_space=pltpu.VMEM),
            pl.BlockSpec(memory_space=pltpu.SMEM),
            pl.BlockSpec(memory_space=pltpu.VMEM),
            pl.BlockSpec(memory_space=pltpu.VMEM),
            pl.BlockSpec(memory_space=pltpu.VMEM),
        ],
        out_specs=[
            pl.BlockSpec(memory_space=pltpu.VMEM),
            pl.BlockSpec(memory_space=pltpu.VMEM),
        ],
    )(z, log_tau, r0, r1, r2)
    return (indices, weights)
